# baseline (speedup 1.0000x reference)
import functools
import math

import jax
import jax.numpy as jnp
from jax import lax
from jax.experimental import pallas as pl
from jax.experimental.pallas import tpu as pltpu
from jax.experimental.pallas import tpu_sc as plsc

VOCAB = 100000
PAD = 0
SMOOTH = 0.1
CONF = 1.0 - SMOOTH
EPS = SMOOTH / (VOCAB - 1)
K0 = CONF * math.log(CONF) + (VOCAB - 2) * EPS * math.log(EPS)
N_TOK = 2048

_NC, _NS = 2, 16
_NW = _NC * _NS
_L = 16
_SC_ROWS = 512
_RPW = _SC_ROWS // _NW
_GPW = _RPW // 8
_CB = 1408
_CALIGNED = 99968
_CPG = _CALIGNED // _CB


def _sc_body(x_ref, tgt_ref, out_ref, t_v, buf0, buf1, acc_v, sem0, sem1):
    wid = lax.axis_index("s") * _NC + lax.axis_index("c")
    rbase = wid * _RPW
    pltpu.sync_copy(tgt_ref.at[pl.ds(rbase, _RPW)], t_v)

    acc = jnp.zeros((_L,), jnp.float32)
    for g in range(_RPW // _L):
        t16 = t_v[pl.ds(g * _L, _L)]
        acc = acc + jnp.where(t16 != PAD, jnp.float32(K0), 0.0)

    zero16 = jnp.zeros((_L,), jnp.int32)

    def start(g, c, buf, sem):
        pltpu.async_copy(
            x_ref.at[pl.ds(rbase + g * 8, 8), pl.ds(c * _CB, _CB)], buf, sem)

    def wait(buf, sem):
        pltpu.make_async_copy(
            x_ref.at[pl.ds(0, 8), pl.ds(0, _CB)], buf, sem).wait()

    def process(g, c, buf, acc):
        for rr in range(8):
            def inner(k, accs):
                base = k * (_L * 8)
                return tuple(accs[u] + buf[rr, pl.ds(base + u * _L, _L)]
                             for u in range(8))

            accs = lax.fori_loop(0, _CB // (_L * 8), inner,
                                 tuple(jnp.zeros((_L,), jnp.float32)
                                       for _ in range(8)))
            csum = (((accs[0] + accs[1]) + (accs[2] + accs[3]))
                    + ((accs[4] + accs[5]) + (accs[6] + accs[7])))
            rloc = jnp.full((_L,), g * 8 + rr, jnp.int32)
            t_sp = plsc.load_gather(t_v, [rloc])
            m = t_sp != PAD
            acc = acc + jnp.where(m, (-EPS) * csum, 0.0)
            local = t_sp - c * _CB
            inb = m & (local >= 0) & (local < _CB)
            rr_sp = jnp.full((_L,), rr, jnp.int32)
            xt = plsc.load_gather(buf, [rr_sp, jnp.clip(local, 0, _CB - 1)])
            acc = acc + jnp.where(inb, (-(CONF - EPS) / _L) * xt, 0.0)
            x0 = plsc.load_gather(buf, [rr_sp, zero16])
            acc = acc + jnp.where(m & (c == 0), (EPS / _L) * x0, 0.0)
        return acc

    for g in range(_GPW):
        start(g, 0, buf0, sem0)

        def pair(cc, acc, g=g):
            c0 = 2 * cc
            start(g, c0 + 1, buf1, sem1)
            wait(buf0, sem0)
            acc = process(g, c0, buf0, acc)
            start(g, c0 + 2, buf0, sem0)
            wait(buf1, sem1)
            acc = process(g, c0 + 1, buf1, acc)
            return acc

        acc = lax.fori_loop(0, (_CPG - 1) // 2, pair, acc)
        wait(buf0, sem0)
        acc = process(g, _CPG - 1, buf0, acc)

    acc_v[...] = acc
    pltpu.sync_copy(acc_v, out_ref.at[wid])


@functools.lru_cache(maxsize=None)
def _make_sc_call():
  return functools.partial(
    pl.kernel,
    out_type=jax.ShapeDtypeStruct((_NW, _L), jnp.float32),
    mesh=plsc.VectorSubcoreMesh(core_axis_name="c", subcore_axis_name="s",
                                num_cores=_NC, num_subcores=_NS),
    scratch_types=[
        pltpu.VMEM((_RPW,), jnp.int32),
        pltpu.VMEM((8, _CB), jnp.float32),
        pltpu.VMEM((8, _CB), jnp.float32),
        pltpu.VMEM((_L,), jnp.float32),
        pltpu.SemaphoreType.DMA,
        pltpu.SemaphoreType.DMA,
    ],
    compiler_params=pltpu.CompilerParams(needs_layout_passes=False,
                                         use_tc_tiling_on_sc=True),
  )(_sc_body)


_R = 16
_TC_ROWS = N_TOK - _SC_ROWS
_NBLK = _TC_ROWS // _R
_NBUF = 6


def _tc_body(t_ref, x_hbm, out_ref, buf, sem):
    i = pl.program_id(0)

    def cp(j, slot):
        return pltpu.make_async_copy(
            x_hbm.at[pl.ds(_SC_ROWS + j * _R, _R), :],
            buf.at[slot], sem.at[slot])

    @pl.when(i == 0)
    def _init():
        out_ref[0, 0] = 0.0
        for k in range(_NBUF - 1):
            cp(k, k).start()

    j = i + _NBUF - 1

    @pl.when(j < _NBLK)
    def _prefetch():
        cp(j, j % _NBUF).start()

    slot = lax.rem(i, _NBUF)
    cp(i, slot).wait()

    x = buf[slot]
    t = t_ref[0]
    col = lax.broadcasted_iota(jnp.int32, (_R, VOCAB), 1)
    w = jnp.where(col == t, jnp.float32(-CONF), jnp.float32(-EPS))
    w = jnp.where(col == 0, 0.0, w)
    xm = jnp.where(t != PAD, x, 0.0)
    cnt = jnp.sum(jnp.where(t != PAD, jnp.float32(K0), 0.0))
    out_ref[0, 0] = out_ref[0, 0] + jnp.sum(xm * w) + cnt


_tc_call = pl.pallas_call(
    _tc_body,
    grid=(_NBLK,),
    in_specs=[
        pl.BlockSpec((1, _R, 1), lambda i: (i, 0, 0)),
        pl.BlockSpec(memory_space=pltpu.HBM),
    ],
    out_specs=pl.BlockSpec((1, 1), lambda i: (0, 0),
                           memory_space=pltpu.SMEM),
    out_shape=jax.ShapeDtypeStruct((1, 1), jnp.float32),
    scratch_shapes=[
        pltpu.VMEM((_NBUF, _R, VOCAB), jnp.float32),
        pltpu.SemaphoreType.DMA((_NBUF,)),
    ],
    compiler_params=pltpu.CompilerParams(
        dimension_semantics=("arbitrary",),
    ),
)


_TAILBLK = _CALIGNED // 128


def _comb_body(t_ref, a_ref, scp_ref, xs_ref, out_ref):
    t = t_ref[...]
    col = _CALIGNED + lax.broadcasted_iota(
        jnp.int32, (_SC_ROWS, 128), 1)
    w = jnp.where(col == t, jnp.float32(-CONF), jnp.float32(-EPS))
    xm = jnp.where((t != PAD) & (col < VOCAB), xs_ref[...], 0.0)
    part = jnp.sum(xm * w)
    out_ref[0, 0] = a_ref[0, 0] + jnp.sum(scp_ref[...]) + part


_comb_call = pl.pallas_call(
    _comb_body,
    grid=(1,),
    in_specs=[
        pl.BlockSpec((_SC_ROWS, 1), lambda i: (0, 0)),
        pl.BlockSpec((1, 1), lambda i: (0, 0), memory_space=pltpu.SMEM),
        pl.BlockSpec((_NW, _L), lambda i: (0, 0)),
        pl.BlockSpec((_SC_ROWS, 128), lambda i: (0, _TAILBLK)),
    ],
    out_specs=pl.BlockSpec((1, 1), lambda i: (0, 0),
                           memory_space=pltpu.SMEM),
    out_shape=jax.ShapeDtypeStruct((1, 1), jnp.float32),
)


def kernel(model_output, target):
    assert model_output.shape == (N_TOK, VOCAB)
    tgt = target.astype(jnp.int32)
    x_sc = model_output[:_SC_ROWS]
    scp = _make_sc_call()(x_sc, tgt)
    t3 = tgt[_SC_ROWS:].reshape(_NBLK, _R, 1)
    tcp = _tc_call(t3, model_output)
    t2 = tgt[:_SC_ROWS].reshape(_SC_ROWS, 1)
    out = _comb_call(t2, tcp, scp, model_output)
    return out[0, 0]


# speedup vs baseline: 1.1031x; 1.1031x over previous
import functools
import math

import jax
import jax.numpy as jnp
from jax import lax
from jax.experimental import pallas as pl
from jax.experimental.pallas import tpu as pltpu
from jax.experimental.pallas import tpu_sc as plsc

VOCAB = 100000
PAD = 0
SMOOTH = 0.1
CONF = 1.0 - SMOOTH
EPS = SMOOTH / (VOCAB - 1)
K0 = CONF * math.log(CONF) + (VOCAB - 2) * EPS * math.log(EPS)
N_TOK = 2048

_NC, _NS = 2, 16
_NW = _NC * _NS
_BPW = N_TOK // _NW
_L = 16


def _sc_body(tgt_ref, out_ref, t_v, acc_v):
    wid = lax.axis_index("s") * _NC + lax.axis_index("c")
    base = wid * _BPW
    pltpu.sync_copy(tgt_ref.at[pl.ds(base, _BPW)], t_v)
    acc = jnp.zeros((_L,), jnp.float32)
    for g in range(_BPW // _L):
        t16 = t_v[pl.ds(g * _L, _L)]
        acc = acc + jnp.where(t16 != PAD, jnp.float32(K0), 0.0)
    acc_v[...] = acc
    pltpu.sync_copy(acc_v, out_ref.at[wid])


@functools.lru_cache(maxsize=None)
def _make_sc_call():
  return functools.partial(
    pl.kernel,
    out_type=jax.ShapeDtypeStruct((_NW, _L), jnp.float32),
    mesh=plsc.VectorSubcoreMesh(core_axis_name="c", subcore_axis_name="s",
                                num_cores=_NC, num_subcores=_NS),
    scratch_types=[
        pltpu.VMEM((_BPW,), jnp.int32),
        pltpu.VMEM((_L,), jnp.float32),
    ],
    compiler_params=pltpu.CompilerParams(needs_layout_passes=False),
  )(_sc_body)


_R = 8
_NBLK = N_TOK // _R
_NBUF = 8


def _tc_body(t_ref, x_hbm, out_ref, buf, sem):
    i = pl.program_id(0)

    def cp(j, slot):
        return pltpu.make_async_copy(
            x_hbm.at[pl.ds(j * _R, _R), :], buf.at[slot], sem.at[slot])

    @pl.when(i == 0)
    def _init():
        out_ref[0, 0] = 0.0
        for k in range(_NBUF - 1):
            cp(k, k).start()

    j = i + _NBUF - 1

    @pl.when(j < _NBLK)
    def _prefetch():
        cp(j, j % _NBUF).start()

    slot = lax.rem(i, _NBUF)
    cp(i, slot).wait()

    x = buf[slot]
    t = t_ref[0]
    col = lax.broadcasted_iota(jnp.int32, (_R, VOCAB), 1)
    w = jnp.where(col == t, jnp.float32(-CONF), jnp.float32(-EPS))
    w = jnp.where(col == 0, 0.0, w)
    xm = jnp.where(t != PAD, x, 0.0)
    out_ref[0, 0] = out_ref[0, 0] + jnp.sum(xm * w)


_tc_call = pl.pallas_call(
    _tc_body,
    grid=(_NBLK,),
    in_specs=[
        pl.BlockSpec((1, _R, 1), lambda i: (i, 0, 0)),
        pl.BlockSpec(memory_space=pltpu.HBM),
    ],
    out_specs=pl.BlockSpec((1, 1), lambda i: (0, 0),
                           memory_space=pltpu.SMEM),
    out_shape=jax.ShapeDtypeStruct((1, 1), jnp.float32),
    scratch_shapes=[
        pltpu.VMEM((_NBUF, _R, VOCAB), jnp.float32),
        pltpu.SemaphoreType.DMA((_NBUF,)),
    ],
    compiler_params=pltpu.CompilerParams(
        dimension_semantics=("arbitrary",),
    ),
)


def _comb_body(a_ref, scp_ref, out_ref):
    out_ref[0, 0] = a_ref[0, 0] + jnp.sum(scp_ref[...])


_comb_call = pl.pallas_call(
    _comb_body,
    grid=(1,),
    in_specs=[
        pl.BlockSpec((1, 1), lambda i: (0, 0), memory_space=pltpu.SMEM),
        pl.BlockSpec((_NW, _L), lambda i: (0, 0)),
    ],
    out_specs=pl.BlockSpec((1, 1), lambda i: (0, 0),
                           memory_space=pltpu.SMEM),
    out_shape=jax.ShapeDtypeStruct((1, 1), jnp.float32),
)


def kernel(model_output, target):
    assert model_output.shape == (N_TOK, VOCAB)
    tgt = target.astype(jnp.int32)
    scp = _make_sc_call()(tgt)
    t3 = tgt.reshape(_NBLK, _R, 1)
    tcp = _tc_call(t3, model_output)
    out = _comb_call(tcp, scp)
    return out[0, 0]


# speedup vs baseline: 1.1349x; 1.0288x over previous
import functools
import math

import jax
import jax.numpy as jnp
from jax import lax
from jax.experimental import pallas as pl
from jax.experimental.pallas import tpu as pltpu
from jax.experimental.pallas import tpu_sc as plsc

VOCAB = 100000
PAD = 0
SMOOTH = 0.1
CONF = 1.0 - SMOOTH
EPS = SMOOTH / (VOCAB - 1)
K0 = CONF * math.log(CONF) + (VOCAB - 2) * EPS * math.log(EPS)
N_TOK = 2048

_NC, _NS = 2, 16
_NW = _NC * _NS
_BPW = N_TOK // _NW
_L = 16


def _sc_body(tgt_ref, out_ref, t_v, acc_v):
    wid = lax.axis_index("s") * _NC + lax.axis_index("c")
    base = wid * _BPW
    pltpu.sync_copy(tgt_ref.at[pl.ds(base, _BPW)], t_v)
    acc = jnp.zeros((_L,), jnp.float32)
    for g in range(_BPW // _L):
        t16 = t_v[pl.ds(g * _L, _L)]
        acc = acc + jnp.where(t16 != PAD, jnp.float32(K0), 0.0)
    acc_v[...] = acc
    pltpu.sync_copy(acc_v, out_ref.at[wid])


@functools.lru_cache(maxsize=None)
def _make_sc_call():
  return functools.partial(
    pl.kernel,
    out_type=jax.ShapeDtypeStruct((_NW, _L), jnp.float32),
    mesh=plsc.VectorSubcoreMesh(core_axis_name="c", subcore_axis_name="s",
                                num_cores=_NC, num_subcores=_NS),
    scratch_types=[
        pltpu.VMEM((_BPW,), jnp.int32),
        pltpu.VMEM((_L,), jnp.float32),
    ],
    compiler_params=pltpu.CompilerParams(needs_layout_passes=False),
  )(_sc_body)


_R = 32
_NBLK = N_TOK // _R
_NBUF = 4


def _tc_body(t_ref, x_hbm, out_ref, buf, sem):
    i = pl.program_id(0)

    def cp(j, slot):
        return pltpu.make_async_copy(
            x_hbm.at[pl.ds(j * _R, _R), :], buf.at[slot], sem.at[slot])

    @pl.when(i == 0)
    def _init():
        out_ref[0, 0] = 0.0
        for k in range(_NBUF - 1):
            cp(k, k).start()

    j = i + _NBUF - 1

    @pl.when(j < _NBLK)
    def _prefetch():
        cp(j, j % _NBUF).start()

    slot = lax.rem(i, _NBUF)
    cp(i, slot).wait()

    x = buf[slot]
    t = t_ref[0]
    col = lax.broadcasted_iota(jnp.int32, (_R, VOCAB), 1)
    w = jnp.where(col == t, jnp.float32(-CONF), jnp.float32(-EPS))
    w = jnp.where(col == 0, 0.0, w)
    xm = jnp.where(t != PAD, x, 0.0)
    out_ref[0, 0] = out_ref[0, 0] + jnp.sum(xm * w)


_tc_call = pl.pallas_call(
    _tc_body,
    grid=(_NBLK,),
    in_specs=[
        pl.BlockSpec((1, _R, 1), lambda i: (i, 0, 0)),
        pl.BlockSpec(memory_space=pltpu.HBM),
    ],
    out_specs=pl.BlockSpec((1, 1), lambda i: (0, 0),
                           memory_space=pltpu.SMEM),
    out_shape=jax.ShapeDtypeStruct((1, 1), jnp.float32),
    scratch_shapes=[
        pltpu.VMEM((_NBUF, _R, VOCAB), jnp.float32),
        pltpu.SemaphoreType.DMA((_NBUF,)),
    ],
    compiler_params=pltpu.CompilerParams(
        dimension_semantics=("arbitrary",),
    ),
)


def _comb_body(a_ref, scp_ref, out_ref):
    out_ref[0, 0] = a_ref[0, 0] + jnp.sum(scp_ref[...])


_comb_call = pl.pallas_call(
    _comb_body,
    grid=(1,),
    in_specs=[
        pl.BlockSpec((1, 1), lambda i: (0, 0), memory_space=pltpu.SMEM),
        pl.BlockSpec((_NW, _L), lambda i: (0, 0)),
    ],
    out_specs=pl.BlockSpec((1, 1), lambda i: (0, 0),
                           memory_space=pltpu.SMEM),
    out_shape=jax.ShapeDtypeStruct((1, 1), jnp.float32),
)


def kernel(model_output, target):
    assert model_output.shape == (N_TOK, VOCAB)
    tgt = target.astype(jnp.int32)
    scp = _make_sc_call()(tgt)
    t3 = tgt.reshape(_NBLK, _R, 1)
    tcp = _tc_call(t3, model_output)
    out = _comb_call(tcp, scp)
    return out[0, 0]
